# baseline (speedup 1.0000x reference)
import jax
import jax.numpy as jnp
from jax import lax
from jax.experimental import pallas as pl
from jax.experimental.pallas import tpu as pltpu
from jax.experimental.pallas import tpu_sc as plsc

N_FIELDS = 26
VOCAB = 100000
EMBED_DIM = 32
BATCH = 16384
PAD_DIM = 128

TOTAL = BATCH * N_FIELDS
NUM_CORES = 2
NUM_SUBCORES = 16
NW = NUM_CORES * NUM_SUBCORES
PER_W = TOTAL // NW
L = 16

IDX_PER_STREAM = 104
STREAMS_PER_CHUNK = 8
CHUNK = IDX_PER_STREAM * STREAMS_PER_CHUNK
N_CHUNKS = PER_W // CHUNK


def _body(x_hbm, tab_hbm, out_hbm, idx_v, rows_v, sem):
    wid = lax.axis_index("s") * NUM_CORES + lax.axis_index("c")
    base = wid * PER_W

    pltpu.sync_copy(x_hbm.at[pl.ds(base, PER_W)], idx_v)

    def ibody(i, _):
        sl = pl.ds(i * L, L)
        pos = (base + i * L) + lax.iota(jnp.int32, 16)
        idx_v[sl] = idx_v[sl] + (pos % N_FIELDS) * VOCAB
        return 0

    lax.fori_loop(0, PER_W // L, ibody, 0, unroll=4)

    def gbody(c, _):
        off = c * CHUNK
        copies = []
        for j in range(STREAMS_PER_CHUNK):
            o = j * IDX_PER_STREAM
            copies.append(
                pltpu.async_copy(
                    tab_hbm.at[idx_v.at[pl.ds(off + o, IDX_PER_STREAM)]],
                    rows_v.at[pl.ds(o, IDX_PER_STREAM)],
                    sem,
                )
            )
        for cp in copies:
            cp.wait()
        pltpu.sync_copy(rows_v, out_hbm.at[pl.ds(base + off, CHUNK)])
        return 0

    lax.fori_loop(0, N_CHUNKS, gbody, 0)


@jax.jit
def _embed(x_flat, tab_pad):
    mesh = plsc.VectorSubcoreMesh(core_axis_name="c", subcore_axis_name="s")
    return pl.kernel(
        _body,
        out_type=jax.ShapeDtypeStruct((TOTAL, PAD_DIM), jnp.float32),
        mesh=mesh,
        scratch_types=[
            pltpu.VMEM((PER_W,), jnp.int32),
            pltpu.VMEM((CHUNK, PAD_DIM), jnp.float32),
            pltpu.SemaphoreType.DMA,
        ],
        compiler_params=pltpu.CompilerParams(use_tc_tiling_on_sc=True),
    )(x_flat, tab_pad)


def kernel(x, tables):
    x_flat = x.astype(jnp.int32).reshape(TOTAL)
    tab_pad = jnp.pad(tables, ((0, 0), (0, 0), (0, PAD_DIM - EMBED_DIM)))
    tab_pad = tab_pad.reshape(N_FIELDS * VOCAB, PAD_DIM)
    out = _embed(x_flat, tab_pad)
    return out[:, :EMBED_DIM].reshape(BATCH, N_FIELDS, EMBED_DIM)


# speedup vs baseline: 1.0352x; 1.0352x over previous
import jax
import jax.numpy as jnp
from jax import lax
from jax.experimental import pallas as pl
from jax.experimental.pallas import tpu as pltpu
from jax.experimental.pallas import tpu_sc as plsc

N_FIELDS = 26
VOCAB = 100000
EMBED_DIM = 32
BATCH = 16384

TOTAL = BATCH * N_FIELDS
NUM_CORES = 2
NUM_SUBCORES = 16
NW = NUM_CORES * NUM_SUBCORES
B_PER_W = BATCH // NW
SUBB = 64
N_SUB = B_PER_W // SUBB
ROW_STRIDE = N_FIELDS * EMBED_DIM
ELS_PER_SUB = SUBB * ROW_STRIDE
ELS_PER_STREAM = 128
STREAM_GROUP = 13
N_GROUPS = ELS_PER_SUB // (ELS_PER_STREAM * STREAM_GROUP)
L = 16


def _body(xt_hbm, tab_hbm, out_hbm, xv, pat, eidx, rows_v, sem):
    wid = lax.axis_index("s") * NUM_CORES + lax.axis_index("c")
    b0 = wid * B_PER_W

    pltpu.sync_copy(xt_hbm.at[:, pl.ds(b0, B_PER_W)], xv)

    def pbody(g, _):
        f = g // 2
        e0 = (g % 2) * L
        pat[pl.ds(g * L, L)] = (f * (EMBED_DIM * VOCAB) + e0 * VOCAB) + lax.iota(
            jnp.int32, 16
        ) * VOCAB
        return 0

    lax.fori_loop(0, ROW_STRIDE // L, pbody, 0)

    def sbody(s, _):
        def jbody(jj, _):
            col = s * SUBB + jj
            zv = lax.iota(jnp.int32, 16) * 0
            colv = zv + col
            jbase = jj * ROW_STRIDE
            for f in range(N_FIELDS):
                fv = zv + f
                xs = plsc.load_gather(xv, [fv, colv])
                for h in range(2):
                    sl = pl.ds(jbase + f * EMBED_DIM + h * L, L)
                    eidx[sl] = xs + pat[pl.ds(f * EMBED_DIM + h * L, L)]
            return 0

        lax.fori_loop(0, SUBB, jbody, 0)

        def gbody(g, _):
            copies = []
            for j in range(STREAM_GROUP):
                o = g * (ELS_PER_STREAM * STREAM_GROUP) + j * ELS_PER_STREAM
                copies.append(
                    pltpu.async_copy(
                        tab_hbm.at[eidx.at[pl.ds(o, ELS_PER_STREAM)]],
                        rows_v.at[pl.ds(o, ELS_PER_STREAM)],
                        sem,
                    )
                )
            for cp in copies:
                cp.wait()
            return 0

        lax.fori_loop(0, N_GROUPS, gbody, 0)
        pltpu.sync_copy(
            rows_v, out_hbm.at[pl.ds((b0 + s * SUBB) * ROW_STRIDE, ELS_PER_SUB)]
        )
        return 0

    lax.fori_loop(0, N_SUB, sbody, 0)


@jax.jit
def _embed(xt, tab_el):
    mesh = plsc.VectorSubcoreMesh(core_axis_name="c", subcore_axis_name="s")
    return pl.kernel(
        _body,
        out_type=jax.ShapeDtypeStruct((TOTAL * EMBED_DIM,), jnp.float32),
        mesh=mesh,
        scratch_types=[
            pltpu.VMEM((N_FIELDS, B_PER_W), jnp.int32),
            pltpu.VMEM((ROW_STRIDE,), jnp.int32),
            pltpu.VMEM((ELS_PER_SUB,), jnp.int32),
            pltpu.VMEM((ELS_PER_SUB,), jnp.float32),
            pltpu.SemaphoreType.DMA,
        ],
        compiler_params=pltpu.CompilerParams(
            use_tc_tiling_on_sc=False, needs_layout_passes=False
        ),
    )(xt, tab_el)


def kernel(x, tables):
    xt = x.astype(jnp.int32).T
    tab_el = tables.transpose(0, 2, 1).reshape(N_FIELDS * EMBED_DIM * VOCAB)
    out = _embed(xt, tab_el)
    return out.reshape(BATCH, N_FIELDS, EMBED_DIM)


# speedup vs baseline: 1.0931x; 1.0559x over previous
import jax
import jax.numpy as jnp
from jax import lax
from jax.experimental import pallas as pl
from jax.experimental.pallas import tpu as pltpu
from jax.experimental.pallas import tpu_sc as plsc

N_FIELDS = 26
VOCAB = 100000
EMBED_DIM = 32
BATCH = 16384

TOTAL = BATCH * N_FIELDS
NUM_CORES = 2
NUM_SUBCORES = 16
NW = NUM_CORES * NUM_SUBCORES
B_PER_W = BATCH // NW
SUB = 64
N_SUB = B_PER_W // SUB
ROWS_PER_SUB = SUB * N_FIELDS
N_OSTREAM = ROWS_PER_SUB // 128
L = 16


def _body(xt_hbm, tab_hbm, out_hbm, xv, oidx, rows_v, gsem, osem):
    wid = lax.axis_index("s") * NUM_CORES + lax.axis_index("c")
    b0 = wid * B_PER_W

    pltpu.sync_copy(xt_hbm.at[:, pl.ds(b0, B_PER_W)], xv)

    def sbody(s, _):
        sb = b0 + s * SUB

        def obody(k, _):
            for t in range(8):
                f = 2 * k + t // 4
                j = (t % 4) * 16 + lax.iota(jnp.int32, 16)
                oidx[k, pl.ds(t * 16, 16)] = (sb + j) * N_FIELDS + f
            return 0

        lax.fori_loop(0, N_OSTREAM, obody, 0)

        gathers = []
        for f in range(N_FIELDS):
            gathers.append(
                pltpu.async_copy(
                    tab_hbm.at[f].at[xv.at[f, pl.ds(s * SUB, SUB)]],
                    rows_v.at[pl.ds(f * SUB, SUB)],
                    gsem,
                )
            )
        for g in gathers:
            g.wait()
        scatters = []
        for k in range(N_OSTREAM):
            scatters.append(
                pltpu.async_copy(
                    rows_v.at[pl.ds(k * 128, 128)],
                    out_hbm.at[oidx.at[k]],
                    osem,
                )
            )
        for sc in scatters:
            sc.wait()
        return 0

    lax.fori_loop(0, N_SUB, sbody, 0)


@jax.jit
def _embed(xt, tables):
    mesh = plsc.VectorSubcoreMesh(core_axis_name="c", subcore_axis_name="s")
    return pl.kernel(
        _body,
        out_type=jax.ShapeDtypeStruct((TOTAL, EMBED_DIM), jnp.float32),
        mesh=mesh,
        scratch_types=[
            pltpu.VMEM((N_FIELDS, B_PER_W), jnp.int32),
            pltpu.VMEM((N_OSTREAM, 128), jnp.int32),
            pltpu.VMEM((ROWS_PER_SUB, EMBED_DIM), jnp.float32),
            pltpu.SemaphoreType.DMA,
            pltpu.SemaphoreType.DMA,
        ],
        compiler_params=pltpu.CompilerParams(use_tc_tiling_on_sc=False),
    )(xt, tables)


def kernel(x, tables):
    xt = x.astype(jnp.int32).T
    out = _embed(xt, tables)
    return out.reshape(BATCH, N_FIELDS, EMBED_DIM)


# speedup vs baseline: 1.0957x; 1.0025x over previous
import jax
import jax.numpy as jnp
from jax import lax
from jax.experimental import pallas as pl
from jax.experimental.pallas import tpu as pltpu
from jax.experimental.pallas import tpu_sc as plsc

N_FIELDS = 26
VOCAB = 100000
EMBED_DIM = 32
BATCH = 16384

TOTAL = BATCH * N_FIELDS
NUM_CORES = 2
NUM_SUBCORES = 16
NW = NUM_CORES * NUM_SUBCORES
B_PER_W = BATCH // NW
SUB = 128
N_SUB = B_PER_W // SUB
ROWS_PER_SUB = SUB * N_FIELDS
N_OSTREAM = N_FIELDS
L = 16


def _body(xt_hbm, tab_hbm, out_hbm, xv, oidx, rows_v, gsem, osem):
    wid = lax.axis_index("s") * NUM_CORES + lax.axis_index("c")
    b0 = wid * B_PER_W

    pltpu.sync_copy(xt_hbm.at[:, pl.ds(b0, B_PER_W)], xv)

    def sbody(s, _):
        sb = b0 + s * SUB

        def obody(f, _):
            for t in range(8):
                j = t * 16 + lax.iota(jnp.int32, 16)
                oidx[f, pl.ds(t * 16, 16)] = (sb + j) * N_FIELDS + f
            return 0

        lax.fori_loop(0, N_OSTREAM, obody, 0)

        gathers = []
        for f in range(N_FIELDS):
            gathers.append(
                pltpu.async_copy(
                    tab_hbm.at[f].at[xv.at[f, pl.ds(s * SUB, SUB)]],
                    rows_v.at[pl.ds(f * SUB, SUB)],
                    gsem,
                )
            )
        for g in gathers:
            g.wait()
        scatters = []
        for k in range(N_OSTREAM):
            scatters.append(
                pltpu.async_copy(
                    rows_v.at[pl.ds(k * SUB, SUB)],
                    out_hbm.at[oidx.at[k]],
                    osem,
                )
            )
        for sc in scatters:
            sc.wait()
        return 0

    lax.fori_loop(0, N_SUB, sbody, 0)


@jax.jit
def _embed(xt, tables):
    mesh = plsc.VectorSubcoreMesh(core_axis_name="c", subcore_axis_name="s")
    return pl.kernel(
        _body,
        out_type=jax.ShapeDtypeStruct((TOTAL, EMBED_DIM), jnp.float32),
        mesh=mesh,
        scratch_types=[
            pltpu.VMEM((N_FIELDS, B_PER_W), jnp.int32),
            pltpu.VMEM((N_OSTREAM, 128), jnp.int32),
            pltpu.VMEM((ROWS_PER_SUB, EMBED_DIM), jnp.float32),
            pltpu.SemaphoreType.DMA,
            pltpu.SemaphoreType.DMA,
        ],
        compiler_params=pltpu.CompilerParams(use_tc_tiling_on_sc=False),
    )(xt, tables)


def kernel(x, tables):
    xt = x.astype(jnp.int32).T
    out = _embed(xt, tables)
    return out.reshape(BATCH, N_FIELDS, EMBED_DIM)
